# Initial kernel scaffold; baseline (speedup 1.0000x reference)
import jax
import jax.numpy as jnp
from jax.experimental import pallas as pl


def kernel(x, edge_index, W1, b1, Wc1, bc1, Wc2, bc2, Wf, bf):
    raise NotImplementedError("write your pallas kernel here")


# speedup vs baseline: 13.1326x; 13.1326x over previous
import functools

import jax
import jax.numpy as jnp
from jax import lax
from jax.experimental import pallas as pl
from jax.experimental.pallas import tpu as pltpu
from jax.experimental.pallas import tpu_sc as plsc

N = 10000
N_PAD = 10240
E = 320000
D_IN = 128
D_H = 128
D_OUT = 64

NC = 2
NS = 16
NW = NC * NS
CHUNK = 128
NCHUNKS = E // CHUNK
ROWS_PER_TILE = N_PAD // NS

_sc_mesh = plsc.VectorSubcoreMesh(core_axis_name="c", subcore_axis_name="s")


@functools.partial(
    pl.kernel,
    out_type=jax.ShapeDtypeStruct((NC, N_PAD), jnp.float32),
    mesh=_sc_mesh,
    scratch_types=[
        pltpu.VMEM((CHUNK,), jnp.int32),
        pltpu.VMEM((CHUNK,), jnp.float32),
        pltpu.VMEM((ROWS_PER_TILE,), jnp.float32),
        pltpu.VMEM_SHARED((N_PAD,), jnp.float32),
    ],
)
def _sc_degree(dst_hbm, out_hbm, dst_v, ones_v, zeros_v, acc_sh):
    c = lax.axis_index("c")
    s = lax.axis_index("s")
    wid = s * NC + c
    for j in range(CHUNK // 16):
        ones_v[pl.ds(j * 16, 16)] = jnp.full((16,), 1.0, jnp.float32)
    for j in range(ROWS_PER_TILE // 16):
        zeros_v[pl.ds(j * 16, 16)] = jnp.zeros((16,), jnp.float32)
    pltpu.sync_copy(zeros_v, acc_sh.at[pl.ds(s * ROWS_PER_TILE,
                                             ROWS_PER_TILE)])
    plsc.subcore_barrier()

    nloc = jnp.where(wid < NCHUNKS - (NCHUNKS // NW) * NW,
                     NCHUNKS // NW + 1, NCHUNKS // NW)

    def body(i, carry):
        base = (wid + i * NW) * CHUNK
        pltpu.sync_copy(dst_hbm.at[pl.ds(base, CHUNK)], dst_v)
        pltpu.sync_copy(ones_v, acc_sh.at[dst_v], add=True)
        return carry

    lax.fori_loop(0, nloc, body, 0)
    plsc.subcore_barrier()
    pltpu.sync_copy(acc_sh.at[pl.ds(s * ROWS_PER_TILE, ROWS_PER_TILE)],
                    out_hbm.at[c, pl.ds(s * ROWS_PER_TILE, ROWS_PER_TILE)])


@functools.partial(
    pl.kernel,
    out_type=jax.ShapeDtypeStruct((NC, N_PAD, D_H), jnp.float32),
    mesh=_sc_mesh,
    scratch_types=[
        pltpu.VMEM((CHUNK,), jnp.int32),
        pltpu.VMEM((CHUNK,), jnp.int32),
        pltpu.VMEM((CHUNK, D_H), jnp.float32),
        pltpu.VMEM((CHUNK, D_H), jnp.float32),
        pltpu.VMEM_SHARED((N_PAD, D_H), jnp.float32),
        pltpu.SemaphoreType.DMA,
    ],
)
def _sc_scatter(hs_hbm, src_hbm, dst_hbm, out_hbm,
                src_v, dst_v, rows_v, zeros_v, acc_sh, sem):
    c = lax.axis_index("c")
    s = lax.axis_index("s")
    wid = s * NC + c

    def zbody(i, carry):
        for j in range(D_H // 16):
            zeros_v[i, pl.ds(j * 16, 16)] = jnp.zeros((16,), jnp.float32)
        return carry

    lax.fori_loop(0, CHUNK, zbody, 0)
    for k in range(ROWS_PER_TILE // CHUNK):
        pltpu.sync_copy(zeros_v,
                        acc_sh.at[pl.ds(s * ROWS_PER_TILE + k * CHUNK,
                                        CHUNK)])
    plsc.subcore_barrier()

    nloc = jnp.where(wid < NCHUNKS - (NCHUNKS // NW) * NW,
                     NCHUNKS // NW + 1, NCHUNKS // NW)

    def body(i, carry):
        base = (wid + i * NW) * CHUNK
        pltpu.sync_copy(src_hbm.at[pl.ds(base, CHUNK)], src_v)
        pltpu.sync_copy(dst_hbm.at[pl.ds(base, CHUNK)], dst_v)
        pltpu.async_copy(hs_hbm.at[src_v], rows_v, sem).wait()
        pltpu.sync_copy(rows_v, acc_sh.at[dst_v], add=True)
        return carry

    lax.fori_loop(0, nloc, body, 0)
    plsc.subcore_barrier()

    for k in range(ROWS_PER_TILE // CHUNK):
        r0 = s * ROWS_PER_TILE + k * CHUNK
        pltpu.sync_copy(acc_sh.at[pl.ds(r0, CHUNK)],
                        out_hbm.at[c].at[pl.ds(r0, CHUNK)])


_BM = 1000


def _tc_a_body(x_ref, w1_ref, b1_ref, wc1_ref, d0_ref, d1_ref,
               h0_ref, hs1_ref, dis_ref):
    h0 = jnp.maximum(
        jnp.dot(x_ref[...], w1_ref[...], preferred_element_type=jnp.float32)
        + b1_ref[...], 0.0)
    deg = d0_ref[...] + d1_ref[...]
    dis = jnp.where(deg > 0.0,
                    1.0 / jnp.sqrt(jnp.where(deg > 0.0, deg, 1.0)), 0.0)
    h0_ref[...] = h0
    dis_ref[...] = dis
    hs1_ref[...] = dis * jnp.dot(h0, wc1_ref[...],
                                 preferred_element_type=jnp.float32)


def _tc_b_body(p0_ref, p1_ref, dis_ref, bc1_ref, wc2_ref, h1_ref, hs2_ref):
    dis = dis_ref[...]
    h1 = dis * (p0_ref[...] + p1_ref[...]) + bc1_ref[...]
    h1_ref[...] = h1
    hs2_ref[...] = dis * jnp.dot(h1, wc2_ref[...],
                                 preferred_element_type=jnp.float32)


def _tc_c_body(q0_ref, q1_ref, dis_ref, bc2_ref, h0_ref, h1_ref,
               wf0_ref, wf1_ref, wf2_ref, bf_ref, out_ref):
    h2 = dis_ref[...] * (q0_ref[...] + q1_ref[...]) + bc2_ref[...]
    out_ref[...] = (
        jnp.dot(h0_ref[...], wf0_ref[...], preferred_element_type=jnp.float32)
        + jnp.dot(h1_ref[...], wf1_ref[...], preferred_element_type=jnp.float32)
        + jnp.dot(h2, wf2_ref[...], preferred_element_type=jnp.float32)
        + bf_ref[...])


def _row_spec(d):
    return pl.BlockSpec((_BM, d), lambda i: (i, 0))


def _full_spec(r, cdim):
    return pl.BlockSpec((r, cdim), lambda i: (0, 0))


def _tc_a(x, W1, b1, Wc1, d0, d1):
    return pl.pallas_call(
        _tc_a_body,
        grid=(N // _BM,),
        in_specs=[_row_spec(D_IN), _full_spec(D_IN, D_H), _full_spec(1, D_H),
                  _full_spec(D_H, D_H), _row_spec(1), _row_spec(1)],
        out_specs=[_row_spec(D_H), _row_spec(D_H), _row_spec(1)],
        out_shape=[jax.ShapeDtypeStruct((N, D_H), jnp.float32),
                   jax.ShapeDtypeStruct((N, D_H), jnp.float32),
                   jax.ShapeDtypeStruct((N, 1), jnp.float32)],
    )(x, W1, b1, Wc1, d0, d1)


def _tc_b(p0, p1, dis, bc1, Wc2):
    return pl.pallas_call(
        _tc_b_body,
        grid=(N // _BM,),
        in_specs=[_row_spec(D_H), _row_spec(D_H), _row_spec(1),
                  _full_spec(1, D_H), _full_spec(D_H, D_H)],
        out_specs=[_row_spec(D_H), _row_spec(D_H)],
        out_shape=[jax.ShapeDtypeStruct((N, D_H), jnp.float32),
                   jax.ShapeDtypeStruct((N, D_H), jnp.float32)],
    )(p0, p1, dis, bc1, Wc2)


def _tc_c(q0, q1, dis, bc2, h0, h1, Wf0, Wf1, Wf2, bf):
    return pl.pallas_call(
        _tc_c_body,
        grid=(N // _BM,),
        in_specs=[_row_spec(D_H), _row_spec(D_H), _row_spec(1),
                  _full_spec(1, D_H), _row_spec(D_H), _row_spec(D_H),
                  _full_spec(D_H, D_OUT), _full_spec(D_H, D_OUT),
                  _full_spec(D_H, D_OUT), _full_spec(1, D_OUT)],
        out_specs=pl.BlockSpec((_BM, D_OUT), lambda i: (i, 0)),
        out_shape=jax.ShapeDtypeStruct((N, D_OUT), jnp.float32),
    )(q0, q1, dis, bc2, h0, h1, Wf0, Wf1, Wf2, bf)


def kernel(x, edge_index, W1, b1, Wc1, bc1, Wc2, bc2, Wf, bf):
    src = edge_index[0].astype(jnp.int32)
    dst = edge_index[1].astype(jnp.int32)

    degp = _sc_degree(dst)
    d0 = degp[0].reshape(N_PAD, 1)
    d1 = degp[1].reshape(N_PAD, 1)

    h0, hs1, dis = _tc_a(x, W1, b1.reshape(1, D_H), Wc1, d0, d1)

    p = _sc_scatter(hs1, src, dst)
    h1, hs2 = _tc_b(p[0], p[1], dis, bc1.reshape(1, D_H), Wc2)

    q = _sc_scatter(hs2, src, dst)
    out = _tc_c(q[0], q[1], dis, bc2.reshape(1, D_H), h0, h1,
                Wf[0:D_H], Wf[D_H:2 * D_H], Wf[2 * D_H:3 * D_H],
                bf.reshape(1, D_OUT))
    return out
